# baseline (speedup 1.0000x reference)
import functools

import jax
import jax.numpy as jnp
from jax import lax
from jax.experimental import pallas as pl
from jax.experimental.pallas import tpu as pltpu
from jax.experimental.pallas import tpu_sc as plsc

D_MODEL = 128
SEQ = 50
LANES = 16
NUM_WORKERS = 32
CHUNK = 128
NBUF = 5
PE_REP = 4


def _positional_encoding(seq, d_model):
    pos = jnp.arange(seq, dtype=jnp.float32)[:, None]
    i = jnp.arange(0, d_model, 2, dtype=jnp.float32)
    div = jnp.exp(-i * (jnp.log(10000.0) / d_model))
    ang = pos * div
    pe = jnp.zeros((seq, d_model), dtype=jnp.float32)
    pe = pe.at[:, 0::2].set(jnp.sin(ang))
    pe = pe.at[:, 1::2].set(jnp.cos(ang))
    return pe


def _make_sc_kernel(n_rows, n_chunks):
    mesh = plsc.VectorSubcoreMesh(core_axis_name="c", subcore_axis_name="s")
    rows_per_w = n_rows // NUM_WORKERS
    n_dreg = D_MODEL // LANES
    assert n_chunks % NBUF == 0
    assert rows_per_w % SEQ == 0

    @functools.partial(
        pl.kernel,
        mesh=mesh,
        out_type=jax.ShapeDtypeStruct((n_rows, D_MODEL), jnp.float32),
        scratch_types=[
            pltpu.VMEM((n_chunks, CHUNK), jnp.int32),
            pltpu.VMEM((PE_REP * SEQ, D_MODEL), jnp.float32),
        ]
        + [pltpu.VMEM((CHUNK, D_MODEL), jnp.float32) for _ in range(NBUF)]
        + [pltpu.SemaphoreType.DMA for _ in range(2 * NBUF)],
    )
    def sc_embed(x_hbm, tab_hbm, pe_hbm, out_hbm, idx_v, pe_v, *bufs_sems):
        bufs = bufs_sems[:NBUF]
        gsem = bufs_sems[NBUF:2 * NBUF]
        ssem = bufs_sems[2 * NBUF:]
        cid = lax.axis_index("c")
        sid = lax.axis_index("s")
        w = sid * 2 + cid
        base = w * rows_per_w
        pltpu.sync_copy(x_hbm.at[w], idx_v)
        pltpu.sync_copy(pe_hbm, pe_v)

        def start_gather(b, c):
            pltpu.async_copy(tab_hbm.at[idx_v.at[c]], bufs[b], gsem[b])

        def wait_gather(b):
            pltpu.make_async_copy(tab_hbm.at[idx_v.at[0]], bufs[b],
                                  gsem[b]).wait()

        def start_scatter(b, c):
            pltpu.async_copy(bufs[b],
                             out_hbm.at[pl.ds(base + c * CHUNK, CHUNK)],
                             ssem[b])

        def wait_scatter(b):
            pltpu.make_async_copy(bufs[b], out_hbm.at[pl.ds(base, CHUNK)],
                                  ssem[b]).wait()

        for b in range(NBUF - 1):
            start_gather(b, b)

        def outer_body(g, carry):
            for b in range(NBUF):
                c = g * NBUF + b
                nb = (b + NBUF - 1) % NBUF
                @pl.when(c >= 1)
                def _():
                    wait_scatter(nb)

                @pl.when(c + NBUF - 1 < n_chunks)
                def _():
                    start_gather(nb, c + NBUF - 1)

                wait_gather(b)

                o = lax.rem(c * CHUNK, SEQ)

                def pe_body(r, carry2):
                    ro = o + r
                    for d in range(n_dreg):
                        sl = pl.ds(d * LANES, LANES)
                        plsc.addupdate(bufs[b].at[r, sl], pe_v[ro, sl])
                    return carry2

                lax.fori_loop(0, CHUNK, pe_body, 0)
                start_scatter(b, c)
            return carry

        lax.fori_loop(0, n_chunks // NBUF, outer_body, 0)
        wait_scatter((n_chunks - 1) % NBUF)

    return sc_embed


def kernel(x, tok_table):
    batch, seq = x.shape
    assert seq == SEQ
    n_rows = batch * seq
    assert n_rows % (NUM_WORKERS * CHUNK) == 0
    n_chunks = n_rows // (NUM_WORKERS * CHUNK)
    x_flat = x.astype(jnp.int32).reshape(NUM_WORKERS, n_chunks, CHUNK)
    pe = jnp.tile(_positional_encoding(SEQ, D_MODEL), (PE_REP, 1))
    sc_embed = _make_sc_kernel(n_rows, n_chunks)
    out = sc_embed(x_flat, tok_table, pe)
    return out.reshape(batch, seq, D_MODEL)


# speedup vs baseline: 2.5003x; 2.5003x over previous
import functools

import jax
import jax.numpy as jnp
from jax import lax
from jax.experimental import pallas as pl
from jax.experimental.pallas import tpu as pltpu
from jax.experimental.pallas import tpu_sc as plsc

D_MODEL = 128
SEQ = 50
LANES = 16
NUM_WORKERS = 32
BATCH_PER_CHUNK = 4
CHUNK = BATCH_PER_CHUNK * SEQ
NBUF = 4


def _positional_encoding(seq, d_model):
    pos = jnp.arange(seq, dtype=jnp.float32)[:, None]
    i = jnp.arange(0, d_model, 2, dtype=jnp.float32)
    div = jnp.exp(-i * (jnp.log(10000.0) / d_model))
    ang = pos * div
    pe = jnp.zeros((seq, d_model), dtype=jnp.float32)
    pe = pe.at[:, 0::2].set(jnp.sin(ang))
    pe = pe.at[:, 1::2].set(jnp.cos(ang))
    return pe


def _make_sc_kernel(batch, n_chunks):
    mesh = plsc.VectorSubcoreMesh(core_axis_name="c", subcore_axis_name="s")
    n_dreg = D_MODEL // LANES
    assert n_chunks % NBUF == 0
    batch_per_w = batch // NUM_WORKERS

    @functools.partial(
        pl.kernel,
        mesh=mesh,
        out_type=jax.ShapeDtypeStruct((batch, SEQ, D_MODEL), jnp.float32),
        scratch_types=[
            pltpu.VMEM((SEQ, D_MODEL), jnp.float32),
        ]
        + [pltpu.VMEM((CHUNK,), jnp.int32) for _ in range(NBUF)]
        + [pltpu.VMEM((CHUNK, D_MODEL), jnp.float32) for _ in range(NBUF)]
        + [pltpu.SemaphoreType.DMA for _ in range(3 * NBUF)],
    )
    def sc_embed(x_hbm, tab_hbm, pe_hbm, out_hbm, pe_v, *bufs_sems):
        ibufs = bufs_sems[:NBUF]
        bufs = bufs_sems[NBUF:2 * NBUF]
        isem = bufs_sems[2 * NBUF:3 * NBUF]
        gsem = bufs_sems[3 * NBUF:4 * NBUF]
        ssem = bufs_sems[4 * NBUF:]
        cid = lax.axis_index("c")
        sid = lax.axis_index("s")
        w = sid * 2 + cid
        pltpu.sync_copy(pe_hbm, pe_v)
        batch_base = w * batch_per_w

        def start_idx(b, c):
            pltpu.async_copy(x_hbm.at[w, c], ibufs[b], isem[b])

        def wait_idx(b):
            pltpu.make_async_copy(x_hbm.at[w, 0], ibufs[b], isem[b]).wait()

        def start_gather(b):
            pltpu.async_copy(tab_hbm.at[ibufs[b]], bufs[b], gsem[b])

        def wait_gather(b):
            pltpu.make_async_copy(tab_hbm.at[ibufs[b]], bufs[b],
                                  gsem[b]).wait()

        def start_scatter(b, c):
            for k in range(BATCH_PER_CHUNK):
                pltpu.async_copy(
                    bufs[b].at[pl.ds(k * SEQ, SEQ)],
                    out_hbm.at[batch_base + c * BATCH_PER_CHUNK + k],
                    ssem[b])

        def wait_scatter(b):
            for _ in range(BATCH_PER_CHUNK):
                pltpu.make_async_copy(bufs[b].at[pl.ds(0, SEQ)],
                                      out_hbm.at[0], ssem[b]).wait()

        for b in range(NBUF):
            start_idx(b, b)
        for b in range(NBUF - 1):
            wait_idx(b)
            start_gather(b)

        def outer_body(g, carry):
            for b in range(NBUF):
                c = g * NBUF + b
                nb = (b + NBUF - 1) % NBUF
                @pl.when(c >= 1)
                def _():
                    wait_scatter(nb)

                @pl.when(c + NBUF - 1 < n_chunks)
                def _():
                    wait_idx(nb)
                    start_gather(nb)

                wait_gather(b)

                @pl.when(c + NBUF < n_chunks)
                def _():
                    start_idx(b, c + NBUF)

                def pe_body(s, carry2):
                    for j in range(BATCH_PER_CHUNK):
                        r = j * SEQ + s
                        for d in range(n_dreg):
                            sl = pl.ds(d * LANES, LANES)
                            plsc.addupdate(bufs[b].at[r, sl], pe_v[s, sl])
                    return carry2

                lax.fori_loop(0, SEQ, pe_body, 0)
                start_scatter(b, c)
            return carry

        lax.fori_loop(0, n_chunks // NBUF, outer_body, 0)
        wait_scatter((n_chunks - 1) % NBUF)

    return sc_embed


def kernel(x, tok_table):
    batch, seq = x.shape
    assert seq == SEQ
    n_rows = batch * seq
    assert n_rows % (NUM_WORKERS * CHUNK) == 0
    n_chunks = n_rows // (NUM_WORKERS * CHUNK)
    x_flat = x.astype(jnp.int32).reshape(NUM_WORKERS, n_chunks, CHUNK)
    pe = _positional_encoding(SEQ, D_MODEL)
    sc_embed = _make_sc_kernel(batch, n_chunks)
    return sc_embed(x_flat, tok_table, pe)
